# baseline (speedup 1.0000x reference)
import functools

import jax
import jax.numpy as jnp
from jax import lax
from jax.experimental import pallas as pl
from jax.experimental.pallas import tpu as pltpu
from jax.experimental.pallas import tpu_sc as plsc

_BB = 128
_GCH = 128


def _argmax_body(x_ref, out_ref):
    x = x_ref[...]
    m = jnp.max(x, axis=-1, keepdims=True)
    ii = jax.lax.broadcasted_iota(jnp.int32, x.shape, 2)
    cand = jnp.where(x == m, ii, x.shape[-1])
    out_ref[...] = jnp.min(cand, axis=-1).T


@functools.partial(jax.jit, static_argnums=())
def _tc_argmax(x):
    B, S, N = x.shape
    return pl.pallas_call(
        _argmax_body,
        grid=(B // _BB,),
        in_specs=[pl.BlockSpec((_BB, S, N), lambda i: (i, 0, 0))],
        out_specs=pl.BlockSpec((S, _BB), lambda i: (0, i)),
        out_shape=jax.ShapeDtypeStruct((S, B), jnp.int32),
    )(x)


@functools.lru_cache(maxsize=None)
def _make_sc_gather(n_rows, D):
    info = plsc.get_sparse_core_info()
    NC, NS = info.num_cores, info.num_subcores
    NW = NC * NS
    rows_per_w = n_rows // NW
    n_ch = rows_per_w // _GCH
    mesh = plsc.VectorSubcoreMesh(core_axis_name="c", subcore_axis_name="s")

    @functools.partial(
        pl.kernel,
        out_type=jax.ShapeDtypeStruct((n_rows, 128), jnp.float32),
        mesh=mesh,
        scratch_types=[
            pltpu.VMEM((n_ch, _GCH), jnp.int32),
            pltpu.VMEM((_GCH, 128), jnp.float32),
            pltpu.SemaphoreType.DMA,
        ],
    )
    def gather(table_hbm, idx_hbm, out_hbm, idx_v, rows_v, sem):
        wid = lax.axis_index("s") * NC + lax.axis_index("c")
        pltpu.sync_copy(idx_hbm.at[wid], idx_v)
        base = wid * rows_per_w
        for g in range(n_ch):
            pltpu.async_copy(table_hbm.at[idx_v.at[g]], rows_v, sem).wait()
            pltpu.sync_copy(rows_v, out_hbm.at[pl.ds(base + g * _GCH, _GCH)])

    return gather


def kernel(x, state_embedding):
    B, S, N = x.shape
    D = state_embedding.shape[1]
    idx_t = _tc_argmax(x)
    n_rows = S * B
    info = plsc.get_sparse_core_info()
    NW = info.num_cores * info.num_subcores
    idx3 = idx_t.reshape(NW, (n_rows // NW) // _GCH, _GCH)
    table_p = jnp.pad(state_embedding, ((0, 0), (0, 128 - D)))
    out = _make_sc_gather(n_rows, D)(table_p, idx3)
    return out[:, :D].reshape(S, B, D)


# speedup vs baseline: 1.0024x; 1.0024x over previous
import functools

import jax
import jax.numpy as jnp
from jax import lax
from jax.experimental import pallas as pl
from jax.experimental.pallas import tpu as pltpu
from jax.experimental.pallas import tpu_sc as plsc

_BB = 256
_GCH = 128


def _argmax_body(x_ref, out_ref):
    x = x_ref[...]
    m = jnp.max(x, axis=-1, keepdims=True)
    ii = jax.lax.broadcasted_iota(jnp.int32, x.shape, 2)
    cand = jnp.where(x == m, ii, x.shape[-1])
    out_ref[...] = jnp.min(cand, axis=-1).T


@functools.partial(jax.jit, static_argnums=())
def _tc_argmax(x):
    B, S, N = x.shape
    return pl.pallas_call(
        _argmax_body,
        grid=(B // _BB,),
        in_specs=[pl.BlockSpec((_BB, S, N), lambda i: (i, 0, 0))],
        out_specs=pl.BlockSpec((S, _BB), lambda i: (0, i)),
        out_shape=jax.ShapeDtypeStruct((S, B), jnp.int32),
    )(x)


@functools.lru_cache(maxsize=None)
def _make_sc_gather(n_rows, D):
    info = plsc.get_sparse_core_info()
    NC, NS = info.num_cores, info.num_subcores
    NW = NC * NS
    rows_per_w = n_rows // NW
    n_ch = rows_per_w // _GCH
    mesh = plsc.VectorSubcoreMesh(core_axis_name="c", subcore_axis_name="s")

    @functools.partial(
        pl.kernel,
        out_type=jax.ShapeDtypeStruct((n_rows, 128), jnp.float32),
        mesh=mesh,
        scratch_types=[
            pltpu.VMEM((n_ch, _GCH), jnp.int32),
            pltpu.VMEM((_GCH, 128), jnp.float32),
            pltpu.SemaphoreType.DMA,
        ],
    )
    def gather(table_hbm, idx_hbm, out_hbm, idx_v, rows_v, sem):
        wid = lax.axis_index("s") * NC + lax.axis_index("c")
        pltpu.sync_copy(idx_hbm.at[wid], idx_v)
        base = wid * rows_per_w
        for g in range(n_ch):
            pltpu.async_copy(table_hbm.at[idx_v.at[g]], rows_v, sem).wait()
            pltpu.sync_copy(rows_v, out_hbm.at[pl.ds(base + g * _GCH, _GCH)])

    return gather


def kernel(x, state_embedding):
    B, S, N = x.shape
    D = state_embedding.shape[1]
    idx_t = _tc_argmax(x)
    n_rows = S * B
    info = plsc.get_sparse_core_info()
    NW = info.num_cores * info.num_subcores
    idx3 = idx_t.reshape(NW, (n_rows // NW) // _GCH, _GCH)
    table_p = jnp.pad(state_embedding, ((0, 0), (0, 128 - D)))
    out = _make_sc_gather(n_rows, D)(table_p, idx3)
    return out[:, :D].reshape(S, B, D)


# speedup vs baseline: 2.9667x; 2.9596x over previous
import functools

import jax
import jax.numpy as jnp
from jax import lax
from jax.experimental import pallas as pl
from jax.experimental.pallas import tpu as pltpu
from jax.experimental.pallas import tpu_sc as plsc

_BBB = 2048
_CH = 640
_DP = 128


def _argmax_body(x_ref, out_ref):
    x = x_ref[...]
    n = x.shape[1]
    m = jnp.max(x, axis=1, keepdims=True)
    ii = lax.broadcasted_iota(jnp.int32, x.shape, 1)
    cand = jnp.where(x == m, ii, n)
    out_ref[...] = jnp.min(cand, axis=1)[:, None, :]


def _tc_argmax(xp):
    S, N, B = xp.shape
    return pl.pallas_call(
        _argmax_body,
        grid=(S, B // _BBB),
        in_specs=[pl.BlockSpec((1, N, _BBB), lambda s, i: (s, 0, i))],
        out_specs=pl.BlockSpec((1, 1, _BBB), lambda s, i: (s, 0, i)),
        out_shape=jax.ShapeDtypeStruct((S, 1, B), jnp.int32),
    )(xp)


@functools.lru_cache(maxsize=None)
def _make_sc_gather(total):
    info = plsc.get_sparse_core_info()
    NC, NS = info.num_cores, info.num_subcores
    per_w = total // (NC * NS)
    mesh = plsc.VectorSubcoreMesh(core_axis_name="c", subcore_axis_name="s")

    @functools.partial(
        pl.kernel,
        out_type=jax.ShapeDtypeStruct((total, _DP), jnp.float32),
        mesh=mesh,
        scratch_types=[
            pltpu.VMEM((_CH,), jnp.int32),
            pltpu.VMEM((_CH, _DP), jnp.float32),
            pltpu.SemaphoreType.DMA,
        ],
    )
    def gather(tab_hbm, idx_hbm, out_hbm, idx_v, rows_v, sem):
        wid = lax.axis_index("s") * NC + lax.axis_index("c")
        base = wid * per_w
        for k in range(per_w // _CH):
            off = base + k * _CH
            pltpu.sync_copy(idx_hbm.at[pl.ds(off, _CH)], idx_v)
            pltpu.async_copy(tab_hbm.at[idx_v], rows_v, sem).wait()
            pltpu.sync_copy(rows_v, out_hbm.at[pl.ds(off, _CH)])

    return gather


def kernel(x, state_embedding):
    B, S, N = x.shape
    D = state_embedding.shape[1]
    xp = jnp.transpose(x, (1, 2, 0))
    idx = _tc_argmax(xp)
    tabp = jnp.pad(state_embedding, ((0, 0), (0, _DP - D)))
    outp = _make_sc_gather(S * B)(tabp, idx.reshape(S * B))
    return outp[:, :D].reshape(S, B, D)
